# baseline (speedup 1.0000x reference)
import functools

import jax
import jax.numpy as jnp
from jax import lax
from jax.experimental import pallas as pl
from jax.experimental.pallas import tpu as pltpu
from jax.experimental.pallas import tpu_sc as plsc

N = 10000
E = 320000
D = 128
NC = 2
NS = 16
NW = NC * NS
EW = E // NW
CHUNK = 80
NCHUNK = EW // CHUNK
NPAD = 10240
NSTRIPE = NPAD // NS

TE = 2000
GE = E // TE
TN = 2000
GN = N // TN

_MESH = plsc.VectorSubcoreMesh(core_axis_name="c", subcore_axis_name="s")
_F32 = jnp.float32


def _ln2d(y, g, b):
    m = jnp.mean(y, axis=-1, keepdims=True)
    d = y - m
    v = jnp.mean(d * d, axis=-1, keepdims=True)
    return d * lax.rsqrt(v + 1e-5) * g + b


def _sc_gather_body(xa_hbm, xb_hbm, dstw_hbm, srcw_hbm, outs_hbm,
                    idxd, idxs, bufa, bufb, gsem, ssem):
    wid = lax.axis_index("s") * NC + lax.axis_index("c")
    pltpu.sync_copy(dstw_hbm.at[wid], idxd)
    pltpu.sync_copy(srcw_hbm.at[wid], idxs)

    def start_gathers(c, p):
        pltpu.async_copy(xa_hbm.at[idxd.at[c]], bufa.at[p], gsem)
        pltpu.async_copy(xb_hbm.at[idxs.at[c]], bufb.at[p], gsem)

    start_gathers(0, 0)

    def body(c, carry):
        p = lax.rem(c, 2)
        q = 1 - p
        base = wid * EW + c * CHUNK

        @pl.when(c > 0)
        def _():
            pltpu.make_async_copy(
                bufa.at[q], outs_hbm.at[pl.ds(0, CHUNK)], ssem).wait()

        @pl.when(c + 1 < NCHUNK)
        def _():
            start_gathers(c + 1, q)

        pltpu.make_async_copy(xa_hbm.at[idxd.at[c]], bufa.at[p], gsem).wait()
        pltpu.make_async_copy(xb_hbm.at[idxs.at[c]], bufb.at[p], gsem).wait()

        def add_row(r, carry2):
            for j in range(D // 16):
                sl = pl.ds(j * 16, 16)
                bufa[p, r, sl] = bufa[p, r, sl] + bufb[p, r, sl]
            return carry2

        lax.fori_loop(0, CHUNK, add_row, 0)
        pltpu.async_copy(bufa.at[p], outs_hbm.at[pl.ds(base, CHUNK)], ssem)
        return carry

    lax.fori_loop(0, NCHUNK, body, 0)
    lastp = lax.rem(NCHUNK - 1, 2)
    pltpu.make_async_copy(bufa.at[lastp], outs_hbm.at[pl.ds(0, CHUNK)], ssem).wait()


_sc_gather = functools.partial(
    pl.kernel,
    _sc_gather_body,
    out_type=jax.ShapeDtypeStruct((E, D), _F32),
    mesh=_MESH,
    scratch_types=[
        pltpu.VMEM((NCHUNK, CHUNK), jnp.int32),
        pltpu.VMEM((NCHUNK, CHUNK), jnp.int32),
        pltpu.VMEM((2, CHUNK, D), _F32),
        pltpu.VMEM((2, CHUNK, D), _F32),
        pltpu.SemaphoreType.DMA,
        pltpu.SemaphoreType.DMA,
    ],
)()


def _sc_scatter_body(msg_hbm, dstw_hbm, zeros_hbm, out_hbm,
                     idxd, buf, acc, lsem):
    cid = lax.axis_index("c")
    sid = lax.axis_index("s")
    wid = sid * NC + cid
    pltpu.sync_copy(dstw_hbm.at[wid], idxd)

    def start_load(c, p):
        base = wid * EW + c * CHUNK
        pltpu.async_copy(msg_hbm.at[pl.ds(base, CHUNK)], buf.at[p], lsem)

    start_load(0, 0)
    pltpu.sync_copy(zeros_hbm.at[pl.ds(sid * NSTRIPE, NSTRIPE)],
                    acc.at[pl.ds(sid * NSTRIPE, NSTRIPE)])
    plsc.subcore_barrier()

    def body(c, carry):
        p = lax.rem(c, 2)
        q = 1 - p

        @pl.when(c + 1 < NCHUNK)
        def _():
            start_load(c + 1, q)

        pltpu.make_async_copy(
            msg_hbm.at[pl.ds(0, CHUNK)], buf.at[p], lsem).wait()
        pltpu.sync_copy(buf.at[p], acc.at[idxd.at[c]], add=True)
        return carry

    lax.fori_loop(0, NCHUNK, body, 0)
    plsc.subcore_barrier()
    pltpu.sync_copy(acc.at[pl.ds(sid * NSTRIPE, NSTRIPE)],
                    out_hbm.at[cid, pl.ds(sid * NSTRIPE, NSTRIPE)])


_sc_scatter = functools.partial(
    pl.kernel,
    _sc_scatter_body,
    out_type=jax.ShapeDtypeStruct((NC, NPAD, D), _F32),
    mesh=_MESH,
    scratch_types=[
        pltpu.VMEM((NCHUNK, CHUNK), jnp.int32),
        pltpu.VMEM((2, CHUNK, D), _F32),
        pltpu.VMEM_SHARED((NPAD, D), _F32),
        pltpu.SemaphoreType.DMA,
    ],
)()


def _full(shape=(D, D)):
    return pl.BlockSpec(shape, lambda i: (0,) * len(shape))


def _enc_node_body(x, a1, c1, a2, c2, a3, c3, g, b, w1a, w1b,
                   xh_o, xa_o, xb_o):
    h = jnp.maximum(jnp.dot(x[...], a1[...]) + c1[...], 0.0)
    h = jnp.maximum(jnp.dot(h, a2[...]) + c2[...], 0.0)
    y = jnp.dot(h, a3[...]) + c3[...]
    xh = _ln2d(y, g[...], b[...])
    xh_o[...] = xh
    xa_o[...] = jnp.dot(xh, w1a[...])
    xb_o[...] = jnp.dot(xh, w1b[...])


def _enc_edge_body(e, a1, c1, a2, c2, a3, c3, g, b, eh_o):
    h = jnp.maximum(jnp.dot(e[...], a1[...]) + c1[...], 0.0)
    h = jnp.maximum(jnp.dot(h, a2[...]) + c2[...], 0.0)
    y = jnp.dot(h, a3[...]) + c3[...]
    eh_o[...] = _ln2d(y, g[...], b[...])


def _edge_body(xs, eh, w1c, b1, w2, b2, w3, b3, g, b, msg_o, ehn_o):
    h1 = jnp.maximum(
        xs[...] + jnp.dot(eh[...], w1c[...]) + b1[...], 0.0)
    h2 = jnp.maximum(jnp.dot(h1, w2[...]) + b2[...], 0.0)
    y = jnp.dot(h2, w3[...]) + b3[...]
    ms = _ln2d(y, g[...], b[...])
    msg_o[...] = ms
    ehn_o[...] = ms + eh[...]


def _node_body(agg2, xh, v1a, v1b, c1, v2, c2, v3, c3, g, b, w1a, w1b,
               xh_o, xa_o, xb_o):
    agg = agg2[0] + agg2[1]
    h = jnp.maximum(
        jnp.dot(agg, v1a[...]) + jnp.dot(xh[...], v1b[...]) + c1[...], 0.0)
    h = jnp.maximum(jnp.dot(h, v2[...]) + c2[...], 0.0)
    y = jnp.dot(h, v3[...]) + c3[...]
    xn = _ln2d(y, g[...], b[...]) + xh[...]
    xh_o[...] = xn
    xa_o[...] = jnp.dot(xn, w1a[...])
    xb_o[...] = jnp.dot(xn, w1b[...])


def _node_dec_body(agg2, xh, v1a, v1b, c1, v2, c2, v3, c3, g, b,
                   d1, e1, d2, e2, d3, e3, out_o):
    agg = agg2[0] + agg2[1]
    h = jnp.maximum(
        jnp.dot(agg, v1a[...]) + jnp.dot(xh[...], v1b[...]) + c1[...], 0.0)
    h = jnp.maximum(jnp.dot(h, v2[...]) + c2[...], 0.0)
    y = jnp.dot(h, v3[...]) + c3[...]
    xn = _ln2d(y, g[...], b[...]) + xh[...]
    h = jnp.maximum(jnp.dot(xn, d1[...]) + e1[...], 0.0)
    h = jnp.maximum(jnp.dot(h, d2[...]) + e2[...], 0.0)
    out_o[...] = jnp.dot(h, d3[...]) + e3[...]


def _row_spec(t, d=D):
    return pl.BlockSpec((t, d), lambda i: (i, 0))


def _mlp3_weight_specs():
    return [_full(), _full((1, D)), _full(), _full((1, D)),
            _full(), _full((1, D)), _full((1, D)), _full((1, D))]


def kernel(x, e, edge_index, params):
    src = edge_index[0]
    dst = edge_index[1]
    dst_w = dst.reshape(NW, NCHUNK, CHUNK)
    src_w = src.reshape(NW, NCHUNK, CHUNK)
    zeros = jnp.zeros((NPAD, D), _F32)

    def mw(mlp):
        out = []
        for l in mlp:
            out.append(l["W"])
            out.append(l["b"].reshape(1, -1))
        return out

    gnn = params["gnn"]
    w1a = [blk["edge_mlp"][0]["W"][0:D] for blk in gnn]
    w1b = [blk["edge_mlp"][0]["W"][D:2 * D] for blk in gnn]
    w1c = [blk["edge_mlp"][0]["W"][2 * D:3 * D] for blk in gnn]
    v1a = [blk["node_mlp"][0]["W"][0:D] for blk in gnn]
    v1b = [blk["node_mlp"][0]["W"][D:2 * D] for blk in gnn]

    en = params["enc_node"]
    xh, xa_t, xb_t = pl.pallas_call(
        _enc_node_body,
        grid=(GN,),
        in_specs=[_row_spec(TN)] + _mlp3_weight_specs() + [_full(), _full()],
        out_specs=[_row_spec(TN)] * 3,
        out_shape=[jax.ShapeDtypeStruct((N, D), _F32)] * 3,
    )(x, *mw(en["mlp"]), en["ln"]["g"].reshape(1, D), en["ln"]["b"].reshape(1, D),
      w1a[0], w1b[0])

    ee = params["enc_edge"]
    eh = pl.pallas_call(
        _enc_edge_body,
        grid=(GE,),
        in_specs=[_row_spec(TE, 16),
                  _full((16, D)), _full((1, D)), _full(), _full((1, D)),
                  _full(), _full((1, D)), _full((1, D)), _full((1, D))],
        out_specs=_row_spec(TE),
        out_shape=jax.ShapeDtypeStruct((E, D), _F32),
    )(e, *mw(ee["mlp"]), ee["ln"]["g"].reshape(1, D), ee["ln"]["b"].reshape(1, D))

    for k, blk in enumerate(gnn):
        xs_g = _sc_gather(xa_t, xb_t, dst_w, src_w)

        em = blk["edge_mlp"]
        msg, eh = pl.pallas_call(
            _edge_body,
            grid=(GE,),
            in_specs=[_row_spec(TE)] * 2 + [
                _full(), _full((1, D)), _full(), _full((1, D)),
                _full(), _full((1, D)), _full((1, D)), _full((1, D))],
            out_specs=[_row_spec(TE)] * 2,
            out_shape=[jax.ShapeDtypeStruct((E, D), _F32)] * 2,
        )(xs_g, eh,
          w1c[k], em[0]["b"].reshape(1, D),
          em[1]["W"], em[1]["b"].reshape(1, D),
          em[2]["W"], em[2]["b"].reshape(1, D),
          blk["edge_ln"]["g"].reshape(1, D), blk["edge_ln"]["b"].reshape(1, D))

        agg2 = _sc_scatter(msg, dst_w, zeros)

        nm = blk["node_mlp"]
        node_w = [v1a[k], v1b[k], nm[0]["b"].reshape(1, D),
                  nm[1]["W"], nm[1]["b"].reshape(1, D),
                  nm[2]["W"], nm[2]["b"].reshape(1, D),
                  blk["node_ln"]["g"].reshape(1, D),
                  blk["node_ln"]["b"].reshape(1, D)]
        node_w_specs = [_full(), _full(), _full((1, D)),
                        _full(), _full((1, D)), _full(), _full((1, D)),
                        _full((1, D)), _full((1, D))]
        if k + 1 < len(gnn):
            xh, xa_t, xb_t = pl.pallas_call(
                _node_body,
                grid=(GN,),
                in_specs=[pl.BlockSpec((NC, TN, D), lambda i: (0, i, 0)),
                          _row_spec(TN)] + node_w_specs + [_full(), _full()],
                out_specs=[_row_spec(TN)] * 3,
                out_shape=[jax.ShapeDtypeStruct((N, D), _F32)] * 3,
            )(agg2, xh, *node_w, w1a[k + 1], w1b[k + 1])
        else:
            dec = params["dec"]
            d3 = jnp.zeros((D, D), _F32).at[:, :3].set(dec[2]["W"])
            e3 = jnp.zeros((1, D), _F32).at[:, :3].set(dec[2]["b"].reshape(1, 3))
            out = pl.pallas_call(
                _node_dec_body,
                grid=(GN,),
                in_specs=[pl.BlockSpec((NC, TN, D), lambda i: (0, i, 0)),
                          _row_spec(TN)] + node_w_specs + [
                              _full(), _full((1, D)), _full(), _full((1, D)),
                              _full(), _full((1, D))],
                out_specs=_row_spec(TN),
                out_shape=jax.ShapeDtypeStruct((N, D), _F32),
            )(agg2, xh, *node_w,
              dec[0]["W"], dec[0]["b"].reshape(1, D),
              dec[1]["W"], dec[1]["b"].reshape(1, D), d3, e3)

    return out[:, :3]


# speedup vs baseline: 1.3726x; 1.3726x over previous
import functools

import jax
import jax.numpy as jnp
from jax import lax
from jax.experimental import pallas as pl
from jax.experimental.pallas import tpu as pltpu
from jax.experimental.pallas import tpu_sc as plsc

N = 10000
E = 320000
D = 128
NC = 2
NS = 16
NW = NC * NS
EW = E // NW
CHUNK = 80
NCHUNK = EW // CHUNK
NPAD = 10240
NSTRIPE = NPAD // NS

TE = 2000
GE = E // TE
TN = 2000
GN = N // TN

_MESH = plsc.VectorSubcoreMesh(core_axis_name="c", subcore_axis_name="s")
_F32 = jnp.float32


def _ln2d(y, g, b):
    m = jnp.mean(y, axis=-1, keepdims=True)
    d = y - m
    v = jnp.mean(d * d, axis=-1, keepdims=True)
    return d * lax.rsqrt(v + 1e-5) * g + b


def _sc_gather_body(xa_hbm, xb_hbm, dstw_hbm, srcw_hbm, outa_hbm, outb_hbm,
                    idxd, idxs, bufa, bufb, gsem, ssem):
    wid = lax.axis_index("s") * NC + lax.axis_index("c")
    pltpu.sync_copy(dstw_hbm.at[wid], idxd)
    pltpu.sync_copy(srcw_hbm.at[wid], idxs)

    def start_gathers(c, p):
        pltpu.async_copy(xa_hbm.at[idxd.at[c]], bufa.at[p], gsem)
        pltpu.async_copy(xb_hbm.at[idxs.at[c]], bufb.at[p], gsem)

    start_gathers(0, 0)

    def body(c, carry):
        p = lax.rem(c, 2)
        q = 1 - p
        base = wid * EW + c * CHUNK

        @pl.when(c > 0)
        def _():
            pltpu.make_async_copy(
                bufa.at[q], outa_hbm.at[pl.ds(0, CHUNK)], ssem).wait()
            pltpu.make_async_copy(
                bufb.at[q], outb_hbm.at[pl.ds(0, CHUNK)], ssem).wait()

        @pl.when(c + 1 < NCHUNK)
        def _():
            start_gathers(c + 1, q)

        pltpu.make_async_copy(xa_hbm.at[idxd.at[c]], bufa.at[p], gsem).wait()
        pltpu.make_async_copy(xb_hbm.at[idxs.at[c]], bufb.at[p], gsem).wait()
        pltpu.async_copy(bufa.at[p], outa_hbm.at[pl.ds(base, CHUNK)], ssem)
        pltpu.async_copy(bufb.at[p], outb_hbm.at[pl.ds(base, CHUNK)], ssem)
        return carry

    lax.fori_loop(0, NCHUNK, body, 0)
    lastp = lax.rem(NCHUNK - 1, 2)
    pltpu.make_async_copy(bufa.at[lastp], outa_hbm.at[pl.ds(0, CHUNK)], ssem).wait()
    pltpu.make_async_copy(bufb.at[lastp], outb_hbm.at[pl.ds(0, CHUNK)], ssem).wait()


_sc_gather = functools.partial(
    pl.kernel,
    _sc_gather_body,
    out_type=[jax.ShapeDtypeStruct((E, D), _F32),
              jax.ShapeDtypeStruct((E, D), _F32)],
    mesh=_MESH,
    scratch_types=[
        pltpu.VMEM((NCHUNK, CHUNK), jnp.int32),
        pltpu.VMEM((NCHUNK, CHUNK), jnp.int32),
        pltpu.VMEM((2, CHUNK, D), _F32),
        pltpu.VMEM((2, CHUNK, D), _F32),
        pltpu.SemaphoreType.DMA,
        pltpu.SemaphoreType.DMA,
    ],
)()


def _sc_scatter_body(msg_hbm, dstw_hbm, zeros_hbm, out_hbm,
                     idxd, buf, acc, lsem, zsem):
    cid = lax.axis_index("c")
    sid = lax.axis_index("s")
    wid = sid * NC + cid
    pltpu.sync_copy(dstw_hbm.at[wid], idxd)

    def start_load(c, p):
        base = wid * EW + c * CHUNK
        pltpu.async_copy(msg_hbm.at[pl.ds(base, CHUNK)], buf.at[p], lsem)

    start_load(0, 0)
    pltpu.sync_copy(zeros_hbm.at[pl.ds(sid * NSTRIPE, NSTRIPE)],
                    acc.at[pl.ds(sid * NSTRIPE, NSTRIPE)])
    plsc.subcore_barrier()

    def body(c, carry):
        p = lax.rem(c, 3)

        @pl.when(c >= 2)
        def _():
            pltpu.make_async_copy(
                buf.at[p], acc.at[idxd.at[c]], zsem).wait()

        @pl.when(c + 1 < NCHUNK)
        def _():
            start_load(c + 1, lax.rem(c + 1, 3))

        pltpu.make_async_copy(
            msg_hbm.at[pl.ds(0, CHUNK)], buf.at[p], lsem).wait()
        pltpu.async_copy(buf.at[p], acc.at[idxd.at[c]], zsem, add=True)
        return carry

    lax.fori_loop(0, NCHUNK, body, 0)
    for _ in range(2):
        pltpu.make_async_copy(
            buf.at[0], acc.at[idxd.at[0]], zsem).wait()
    plsc.subcore_barrier()
    pltpu.sync_copy(acc.at[pl.ds(sid * NSTRIPE, NSTRIPE)],
                    out_hbm.at[cid, pl.ds(sid * NSTRIPE, NSTRIPE)])


_sc_scatter = functools.partial(
    pl.kernel,
    _sc_scatter_body,
    out_type=jax.ShapeDtypeStruct((NC, NPAD, D), _F32),
    mesh=_MESH,
    scratch_types=[
        pltpu.VMEM((NCHUNK, CHUNK), jnp.int32),
        pltpu.VMEM((3, CHUNK, D), _F32),
        pltpu.VMEM_SHARED((NPAD, D), _F32),
        pltpu.SemaphoreType.DMA,
        pltpu.SemaphoreType.DMA,
    ],
)()


def _full(shape=(D, D)):
    return pl.BlockSpec(shape, lambda i: (0,) * len(shape))


def _enc_node_body(x, a1, c1, a2, c2, a3, c3, g, b, w1a, w1b,
                   xh_o, xa_o, xb_o):
    h = jnp.maximum(jnp.dot(x[...], a1[...]) + c1[...], 0.0)
    h = jnp.maximum(jnp.dot(h, a2[...]) + c2[...], 0.0)
    y = jnp.dot(h, a3[...]) + c3[...]
    xh = _ln2d(y, g[...], b[...])
    xh_o[...] = xh
    xa_o[...] = jnp.dot(xh, w1a[...])
    xb_o[...] = jnp.dot(xh, w1b[...])


def _enc_edge_body(e, a1, c1, a2, c2, a3, c3, g, b, eh_o):
    h = jnp.maximum(jnp.dot(e[...], a1[...]) + c1[...], 0.0)
    h = jnp.maximum(jnp.dot(h, a2[...]) + c2[...], 0.0)
    y = jnp.dot(h, a3[...]) + c3[...]
    eh_o[...] = _ln2d(y, g[...], b[...])


def _edge_body(xa, xb, eh, w1c, b1, w2, b2, w3, b3, g, b, msg_o, ehn_o):
    h1 = jnp.maximum(
        xa[...] + xb[...] + jnp.dot(eh[...], w1c[...]) + b1[...], 0.0)
    h2 = jnp.maximum(jnp.dot(h1, w2[...]) + b2[...], 0.0)
    y = jnp.dot(h2, w3[...]) + b3[...]
    ms = _ln2d(y, g[...], b[...])
    msg_o[...] = ms
    ehn_o[...] = ms + eh[...]


def _node_body(agg2, xh, v1a, v1b, c1, v2, c2, v3, c3, g, b, w1a, w1b,
               xh_o, xa_o, xb_o):
    agg = agg2[0] + agg2[1]
    h = jnp.maximum(
        jnp.dot(agg, v1a[...]) + jnp.dot(xh[...], v1b[...]) + c1[...], 0.0)
    h = jnp.maximum(jnp.dot(h, v2[...]) + c2[...], 0.0)
    y = jnp.dot(h, v3[...]) + c3[...]
    xn = _ln2d(y, g[...], b[...]) + xh[...]
    xh_o[...] = xn
    xa_o[...] = jnp.dot(xn, w1a[...])
    xb_o[...] = jnp.dot(xn, w1b[...])


def _node_dec_body(agg2, xh, v1a, v1b, c1, v2, c2, v3, c3, g, b,
                   d1, e1, d2, e2, d3, e3, out_o):
    agg = agg2[0] + agg2[1]
    h = jnp.maximum(
        jnp.dot(agg, v1a[...]) + jnp.dot(xh[...], v1b[...]) + c1[...], 0.0)
    h = jnp.maximum(jnp.dot(h, v2[...]) + c2[...], 0.0)
    y = jnp.dot(h, v3[...]) + c3[...]
    xn = _ln2d(y, g[...], b[...]) + xh[...]
    h = jnp.maximum(jnp.dot(xn, d1[...]) + e1[...], 0.0)
    h = jnp.maximum(jnp.dot(h, d2[...]) + e2[...], 0.0)
    out_o[...] = jnp.dot(h, d3[...]) + e3[...]


def _row_spec(t, d=D):
    return pl.BlockSpec((t, d), lambda i: (i, 0))


def _mlp3_weight_specs():
    return [_full(), _full((1, D)), _full(), _full((1, D)),
            _full(), _full((1, D)), _full((1, D)), _full((1, D))]


def kernel(x, e, edge_index, params):
    src = edge_index[0]
    dst = edge_index[1]
    dst_w = dst.reshape(NW, NCHUNK, CHUNK)
    src_w = src.reshape(NW, NCHUNK, CHUNK)
    zeros = jnp.zeros((NPAD, D), _F32)

    def mw(mlp):
        out = []
        for l in mlp:
            out.append(l["W"])
            out.append(l["b"].reshape(1, -1))
        return out

    gnn = params["gnn"]
    w1a = [blk["edge_mlp"][0]["W"][0:D] for blk in gnn]
    w1b = [blk["edge_mlp"][0]["W"][D:2 * D] for blk in gnn]
    w1c = [blk["edge_mlp"][0]["W"][2 * D:3 * D] for blk in gnn]
    v1a = [blk["node_mlp"][0]["W"][0:D] for blk in gnn]
    v1b = [blk["node_mlp"][0]["W"][D:2 * D] for blk in gnn]

    en = params["enc_node"]
    xh, xa_t, xb_t = pl.pallas_call(
        _enc_node_body,
        grid=(GN,),
        in_specs=[_row_spec(TN)] + _mlp3_weight_specs() + [_full(), _full()],
        out_specs=[_row_spec(TN)] * 3,
        out_shape=[jax.ShapeDtypeStruct((N, D), _F32)] * 3,
    )(x, *mw(en["mlp"]), en["ln"]["g"].reshape(1, D), en["ln"]["b"].reshape(1, D),
      w1a[0], w1b[0])

    ee = params["enc_edge"]
    eh = pl.pallas_call(
        _enc_edge_body,
        grid=(GE,),
        in_specs=[_row_spec(TE, 16),
                  _full((16, D)), _full((1, D)), _full(), _full((1, D)),
                  _full(), _full((1, D)), _full((1, D)), _full((1, D))],
        out_specs=_row_spec(TE),
        out_shape=jax.ShapeDtypeStruct((E, D), _F32),
    )(e, *mw(ee["mlp"]), ee["ln"]["g"].reshape(1, D), ee["ln"]["b"].reshape(1, D))

    for k, blk in enumerate(gnn):
        xa_g, xb_g = _sc_gather(xa_t, xb_t, dst_w, src_w)

        em = blk["edge_mlp"]
        msg, eh = pl.pallas_call(
            _edge_body,
            grid=(GE,),
            in_specs=[_row_spec(TE)] * 3 + [
                _full(), _full((1, D)), _full(), _full((1, D)),
                _full(), _full((1, D)), _full((1, D)), _full((1, D))],
            out_specs=[_row_spec(TE)] * 2,
            out_shape=[jax.ShapeDtypeStruct((E, D), _F32)] * 2,
        )(xa_g, xb_g, eh,
          w1c[k], em[0]["b"].reshape(1, D),
          em[1]["W"], em[1]["b"].reshape(1, D),
          em[2]["W"], em[2]["b"].reshape(1, D),
          blk["edge_ln"]["g"].reshape(1, D), blk["edge_ln"]["b"].reshape(1, D))

        agg2 = _sc_scatter(msg, dst_w, zeros)

        nm = blk["node_mlp"]
        node_w = [v1a[k], v1b[k], nm[0]["b"].reshape(1, D),
                  nm[1]["W"], nm[1]["b"].reshape(1, D),
                  nm[2]["W"], nm[2]["b"].reshape(1, D),
                  blk["node_ln"]["g"].reshape(1, D),
                  blk["node_ln"]["b"].reshape(1, D)]
        node_w_specs = [_full(), _full(), _full((1, D)),
                        _full(), _full((1, D)), _full(), _full((1, D)),
                        _full((1, D)), _full((1, D))]
        if k + 1 < len(gnn):
            xh, xa_t, xb_t = pl.pallas_call(
                _node_body,
                grid=(GN,),
                in_specs=[pl.BlockSpec((NC, TN, D), lambda i: (0, i, 0)),
                          _row_spec(TN)] + node_w_specs + [_full(), _full()],
                out_specs=[_row_spec(TN)] * 3,
                out_shape=[jax.ShapeDtypeStruct((N, D), _F32)] * 3,
            )(agg2, xh, *node_w, w1a[k + 1], w1b[k + 1])
        else:
            dec = params["dec"]
            d3 = jnp.zeros((D, D), _F32).at[:, :3].set(dec[2]["W"])
            e3 = jnp.zeros((1, D), _F32).at[:, :3].set(dec[2]["b"].reshape(1, 3))
            out = pl.pallas_call(
                _node_dec_body,
                grid=(GN,),
                in_specs=[pl.BlockSpec((NC, TN, D), lambda i: (0, i, 0)),
                          _row_spec(TN)] + node_w_specs + [
                              _full(), _full((1, D)), _full(), _full((1, D)),
                              _full(), _full((1, D))],
                out_specs=_row_spec(TN),
                out_shape=jax.ShapeDtypeStruct((N, D), _F32),
            )(agg2, xh, *node_w,
              dec[0]["W"], dec[0]["b"].reshape(1, D),
              dec[1]["W"], dec[1]["b"].reshape(1, D), d3, e3)

    return out[:, :3]


# speedup vs baseline: 1.3773x; 1.0034x over previous
import functools

import jax
import jax.numpy as jnp
from jax import lax
from jax.experimental import pallas as pl
from jax.experimental.pallas import tpu as pltpu
from jax.experimental.pallas import tpu_sc as plsc

N = 10000
E = 320000
D = 128
NC = 2
NS = 16
NW = NC * NS
NH = 2
EH = E // NH
EWH = EH // NW
CHUNK = 40
NCHUNK = EWH // CHUNK
NPAD = 10240
NSTRIPE = NPAD // NS

TE = 2000
GEH = EH // TE
TN = 2000
GN = N // TN

_MESH = plsc.VectorSubcoreMesh(core_axis_name="c", subcore_axis_name="s")
_F32 = jnp.float32


def _ln2d(y, g, b):
    m = jnp.mean(y, axis=-1, keepdims=True)
    d = y - m
    v = jnp.mean(d * d, axis=-1, keepdims=True)
    return d * lax.rsqrt(v + 1e-5) * g + b


def _sc_gather_body(xa_hbm, xb_hbm, dstw_hbm, srcw_hbm, outa_hbm, outb_hbm,
                    idxd, idxs, bufa, bufb, gsem, ssem):
    wid = lax.axis_index("s") * NC + lax.axis_index("c")
    pltpu.sync_copy(dstw_hbm.at[wid], idxd)
    pltpu.sync_copy(srcw_hbm.at[wid], idxs)

    def start_gathers(c, p):
        pltpu.async_copy(xa_hbm.at[idxd.at[c]], bufa.at[p], gsem)
        pltpu.async_copy(xb_hbm.at[idxs.at[c]], bufb.at[p], gsem)

    start_gathers(0, 0)

    def body(c, carry):
        p = lax.rem(c, 2)
        q = 1 - p
        base = wid * EWH + c * CHUNK

        @pl.when(c > 0)
        def _():
            pltpu.make_async_copy(
                bufa.at[q], outa_hbm.at[pl.ds(0, CHUNK)], ssem).wait()
            pltpu.make_async_copy(
                bufb.at[q], outb_hbm.at[pl.ds(0, CHUNK)], ssem).wait()

        @pl.when(c + 1 < NCHUNK)
        def _():
            start_gathers(c + 1, q)

        pltpu.make_async_copy(xa_hbm.at[idxd.at[c]], bufa.at[p], gsem).wait()
        pltpu.make_async_copy(xb_hbm.at[idxs.at[c]], bufb.at[p], gsem).wait()
        pltpu.async_copy(bufa.at[p], outa_hbm.at[pl.ds(base, CHUNK)], ssem)
        pltpu.async_copy(bufb.at[p], outb_hbm.at[pl.ds(base, CHUNK)], ssem)
        return carry

    lax.fori_loop(0, NCHUNK, body, 0)
    lastp = lax.rem(NCHUNK - 1, 2)
    pltpu.make_async_copy(bufa.at[lastp], outa_hbm.at[pl.ds(0, CHUNK)], ssem).wait()
    pltpu.make_async_copy(bufb.at[lastp], outb_hbm.at[pl.ds(0, CHUNK)], ssem).wait()


_sc_gather = functools.partial(
    pl.kernel,
    _sc_gather_body,
    out_type=[jax.ShapeDtypeStruct((EH, D), _F32),
              jax.ShapeDtypeStruct((EH, D), _F32)],
    mesh=_MESH,
    scratch_types=[
        pltpu.VMEM((NCHUNK, CHUNK), jnp.int32),
        pltpu.VMEM((NCHUNK, CHUNK), jnp.int32),
        pltpu.VMEM((2, CHUNK, D), _F32),
        pltpu.VMEM((2, CHUNK, D), _F32),
        pltpu.SemaphoreType.DMA,
        pltpu.SemaphoreType.DMA,
    ],
)()


def _sc_scatter_body(msg_hbm, dstw_hbm, zeros_hbm, out_hbm,
                     idxd, buf, acc, lsem, zsem):
    cid = lax.axis_index("c")
    sid = lax.axis_index("s")
    wid = sid * NC + cid
    pltpu.sync_copy(dstw_hbm.at[wid], idxd)

    def start_load(c, p):
        base = wid * EWH + c * CHUNK
        pltpu.async_copy(msg_hbm.at[pl.ds(base, CHUNK)], buf.at[p], lsem)

    start_load(0, 0)
    pltpu.sync_copy(zeros_hbm.at[pl.ds(sid * NSTRIPE, NSTRIPE)],
                    acc.at[pl.ds(sid * NSTRIPE, NSTRIPE)])
    plsc.subcore_barrier()

    def body(c, carry):
        p = lax.rem(c, 3)

        @pl.when(c >= 2)
        def _():
            pltpu.make_async_copy(
                buf.at[p], acc.at[idxd.at[c]], zsem).wait()

        @pl.when(c + 1 < NCHUNK)
        def _():
            start_load(c + 1, lax.rem(c + 1, 3))

        pltpu.make_async_copy(
            msg_hbm.at[pl.ds(0, CHUNK)], buf.at[p], lsem).wait()
        pltpu.async_copy(buf.at[p], acc.at[idxd.at[c]], zsem, add=True)
        return carry

    lax.fori_loop(0, NCHUNK, body, 0)
    for _ in range(2):
        pltpu.make_async_copy(
            buf.at[0], acc.at[idxd.at[0]], zsem).wait()
    plsc.subcore_barrier()
    pltpu.sync_copy(acc.at[pl.ds(sid * NSTRIPE, NSTRIPE)],
                    out_hbm.at[cid, pl.ds(sid * NSTRIPE, NSTRIPE)])


_sc_scatter = functools.partial(
    pl.kernel,
    _sc_scatter_body,
    out_type=jax.ShapeDtypeStruct((NC, NPAD, D), _F32),
    mesh=_MESH,
    scratch_types=[
        pltpu.VMEM((NCHUNK, CHUNK), jnp.int32),
        pltpu.VMEM((3, CHUNK, D), _F32),
        pltpu.VMEM_SHARED((NPAD, D), _F32),
        pltpu.SemaphoreType.DMA,
        pltpu.SemaphoreType.DMA,
    ],
)()


def _full(shape=(D, D)):
    return pl.BlockSpec(shape, lambda i: (0,) * len(shape))


def _enc_node_body(x, a1, c1, a2, c2, a3, c3, g, b, w1a, w1b,
                   xh_o, xa_o, xb_o):
    h = jnp.maximum(jnp.dot(x[...], a1[...]) + c1[...], 0.0)
    h = jnp.maximum(jnp.dot(h, a2[...]) + c2[...], 0.0)
    y = jnp.dot(h, a3[...]) + c3[...]
    xh = _ln2d(y, g[...], b[...])
    xh_o[...] = xh
    xa_o[...] = jnp.dot(xh, w1a[...])
    xb_o[...] = jnp.dot(xh, w1b[...])


def _enc_edge_body(e, a1, c1, a2, c2, a3, c3, g, b, eh_o):
    h = jnp.maximum(jnp.dot(e[...], a1[...]) + c1[...], 0.0)
    h = jnp.maximum(jnp.dot(h, a2[...]) + c2[...], 0.0)
    y = jnp.dot(h, a3[...]) + c3[...]
    eh_o[...] = _ln2d(y, g[...], b[...])


def _edge_body(xa, xb, eh, w1c, b1, w2, b2, w3, b3, g, b, msg_o, ehn_o):
    h1 = jnp.maximum(
        xa[...] + xb[...] + jnp.dot(eh[...], w1c[...]) + b1[...], 0.0)
    h2 = jnp.maximum(jnp.dot(h1, w2[...]) + b2[...], 0.0)
    y = jnp.dot(h2, w3[...]) + b3[...]
    ms = _ln2d(y, g[...], b[...])
    msg_o[...] = ms
    ehn_o[...] = ms + eh[...]


def _node_body(agg0, agg1, xh, v1a, v1b, c1, v2, c2, v3, c3, g, b, w1a, w1b,
               xh_o, xa_o, xb_o):
    agg = agg0[0] + agg0[1] + agg1[0] + agg1[1]
    h = jnp.maximum(
        jnp.dot(agg, v1a[...]) + jnp.dot(xh[...], v1b[...]) + c1[...], 0.0)
    h = jnp.maximum(jnp.dot(h, v2[...]) + c2[...], 0.0)
    y = jnp.dot(h, v3[...]) + c3[...]
    xn = _ln2d(y, g[...], b[...]) + xh[...]
    xh_o[...] = xn
    xa_o[...] = jnp.dot(xn, w1a[...])
    xb_o[...] = jnp.dot(xn, w1b[...])


def _node_dec_body(agg0, agg1, xh, v1a, v1b, c1, v2, c2, v3, c3, g, b,
                   d1, e1, d2, e2, d3, e3, out_o):
    agg = agg0[0] + agg0[1] + agg1[0] + agg1[1]
    h = jnp.maximum(
        jnp.dot(agg, v1a[...]) + jnp.dot(xh[...], v1b[...]) + c1[...], 0.0)
    h = jnp.maximum(jnp.dot(h, v2[...]) + c2[...], 0.0)
    y = jnp.dot(h, v3[...]) + c3[...]
    xn = _ln2d(y, g[...], b[...]) + xh[...]
    h = jnp.maximum(jnp.dot(xn, d1[...]) + e1[...], 0.0)
    h = jnp.maximum(jnp.dot(h, d2[...]) + e2[...], 0.0)
    out_o[...] = jnp.dot(h, d3[...]) + e3[...]


def _row_spec(t, d=D):
    return pl.BlockSpec((t, d), lambda i: (i, 0))


def _mlp3_weight_specs():
    return [_full(), _full((1, D)), _full(), _full((1, D)),
            _full(), _full((1, D)), _full((1, D)), _full((1, D))]


_EDGE_W_SPECS = [_full(), _full((1, D)), _full(), _full((1, D)),
                 _full(), _full((1, D)), _full((1, D)), _full((1, D))]
_NODE_W_SPECS = [_full(), _full(), _full((1, D)),
                 _full(), _full((1, D)), _full(), _full((1, D)),
                 _full((1, D)), _full((1, D))]
_AGG_SPEC = pl.BlockSpec((NC, TN, D), lambda i: (0, i, 0))


def kernel(x, e, edge_index, params):
    src = edge_index[0]
    dst = edge_index[1]
    dst_w = [dst[h * EH:(h + 1) * EH].reshape(NW, NCHUNK, CHUNK)
             for h in range(NH)]
    src_w = [src[h * EH:(h + 1) * EH].reshape(NW, NCHUNK, CHUNK)
             for h in range(NH)]
    zeros = jnp.zeros((NPAD, D), _F32)

    def mw(mlp):
        out = []
        for l in mlp:
            out.append(l["W"])
            out.append(l["b"].reshape(1, -1))
        return out

    gnn = params["gnn"]
    w1a = [blk["edge_mlp"][0]["W"][0:D] for blk in gnn]
    w1b = [blk["edge_mlp"][0]["W"][D:2 * D] for blk in gnn]
    w1c = [blk["edge_mlp"][0]["W"][2 * D:3 * D] for blk in gnn]
    v1a = [blk["node_mlp"][0]["W"][0:D] for blk in gnn]
    v1b = [blk["node_mlp"][0]["W"][D:2 * D] for blk in gnn]

    en = params["enc_node"]
    xh, xa_t, xb_t = pl.pallas_call(
        _enc_node_body,
        grid=(GN,),
        in_specs=[_row_spec(TN)] + _mlp3_weight_specs() + [_full(), _full()],
        out_specs=[_row_spec(TN)] * 3,
        out_shape=[jax.ShapeDtypeStruct((N, D), _F32)] * 3,
    )(x, *mw(en["mlp"]), en["ln"]["g"].reshape(1, D), en["ln"]["b"].reshape(1, D),
      w1a[0], w1b[0])

    ee = params["enc_edge"]
    eh = [pl.pallas_call(
        _enc_edge_body,
        grid=(GEH,),
        in_specs=[_row_spec(TE, 16),
                  _full((16, D)), _full((1, D)), _full(), _full((1, D)),
                  _full(), _full((1, D)), _full((1, D)), _full((1, D))],
        out_specs=_row_spec(TE),
        out_shape=jax.ShapeDtypeStruct((EH, D), _F32),
    )(e[h * EH:(h + 1) * EH], *mw(ee["mlp"]),
      ee["ln"]["g"].reshape(1, D), ee["ln"]["b"].reshape(1, D))
        for h in range(NH)]

    for k, blk in enumerate(gnn):
        em = blk["edge_mlp"]
        edge_w = [w1c[k], em[0]["b"].reshape(1, D),
                  em[1]["W"], em[1]["b"].reshape(1, D),
                  em[2]["W"], em[2]["b"].reshape(1, D),
                  blk["edge_ln"]["g"].reshape(1, D),
                  blk["edge_ln"]["b"].reshape(1, D)]

        gath = [_sc_gather(xa_t, xb_t, dst_w[h], src_w[h]) for h in range(NH)]
        aggs = []
        for h in range(NH):
            xa_g, xb_g = gath[h]
            m, ehn = pl.pallas_call(
                _edge_body,
                grid=(GEH,),
                in_specs=[_row_spec(TE)] * 3 + _EDGE_W_SPECS,
                out_specs=[_row_spec(TE)] * 2,
                out_shape=[jax.ShapeDtypeStruct((EH, D), _F32)] * 2,
            )(xa_g, xb_g, eh[h], *edge_w)
            eh[h] = ehn
            aggs.append(_sc_scatter(m, dst_w[h], zeros))

        nm = blk["node_mlp"]
        node_w = [v1a[k], v1b[k], nm[0]["b"].reshape(1, D),
                  nm[1]["W"], nm[1]["b"].reshape(1, D),
                  nm[2]["W"], nm[2]["b"].reshape(1, D),
                  blk["node_ln"]["g"].reshape(1, D),
                  blk["node_ln"]["b"].reshape(1, D)]
        if k + 1 < len(gnn):
            xh, xa_t, xb_t = pl.pallas_call(
                _node_body,
                grid=(GN,),
                in_specs=[_AGG_SPEC, _AGG_SPEC, _row_spec(TN)]
                + _NODE_W_SPECS + [_full(), _full()],
                out_specs=[_row_spec(TN)] * 3,
                out_shape=[jax.ShapeDtypeStruct((N, D), _F32)] * 3,
            )(aggs[0], aggs[1], xh, *node_w, w1a[k + 1], w1b[k + 1])
        else:
            dec = params["dec"]
            d3 = jnp.zeros((D, D), _F32).at[:, :3].set(dec[2]["W"])
            e3 = jnp.zeros((1, D), _F32).at[:, :3].set(dec[2]["b"].reshape(1, 3))
            out = pl.pallas_call(
                _node_dec_body,
                grid=(GN,),
                in_specs=[_AGG_SPEC, _AGG_SPEC, _row_spec(TN)]
                + _NODE_W_SPECS + [
                    _full(), _full((1, D)), _full(), _full((1, D)),
                    _full(), _full((1, D))],
                out_specs=_row_spec(TN),
                out_shape=jax.ShapeDtypeStruct((N, D), _F32),
            )(aggs[0], aggs[1], xh, *node_w,
              dec[0]["W"], dec[0]["b"].reshape(1, D),
              dec[1]["W"], dec[1]["b"].reshape(1, D), d3, e3)

    return out[:, :3]
